# baseline (speedup 1.0000x reference)
import functools

import numpy as np
import jax
import jax.numpy as jnp
from jax import lax
from jax.experimental import pallas as pl
from jax.experimental.pallas import tpu as pltpu

_HASH_SEED = 1
_DEP = 5
_SLOTS = 5120
_PRIME = 2147483647
_BATCH = 4096
_ROWS = _DEP * _BATCH

_NCORE = 2
_CB = 64
_NBUF = 8
_NCHUNK = _ROWS // _CB
_CPC = _NCHUNK // _NCORE


def _hash_tables():
    rng = np.random.RandomState(_HASH_SEED)
    A = rng.randint(1, _PRIME, size=(_DEP,)).astype(np.int64)
    B = rng.randint(0, _PRIME, size=(_DEP,)).astype(np.int64)
    v = np.arange(1024, dtype=np.int64)
    T0 = (A[:, None] * v[None, :] + B[:, None]) % _PRIME
    T1 = (A[:, None] * 1024 * v[None, :]) % _PRIME
    return T0.astype(np.uint32), T1.astype(np.uint32)


_T0, _T1 = _hash_tables()


def _onehot_stream_body(slot_hbm, val_hbm, out_hbm, slot_v, val_v, ring_ref,
                        sems, insem):
    core = lax.axis_index("core").astype(jnp.int32)
    base = core * jnp.int32(_CPC)

    pltpu.make_async_copy(slot_hbm.at[pl.ds(base, _CPC)], slot_v, insem).start()
    pltpu.make_async_copy(slot_hbm.at[pl.ds(base, _CPC)], slot_v, insem).wait()
    pltpu.make_async_copy(val_hbm.at[pl.ds(base, _CPC)], val_v, insem).start()
    pltpu.make_async_copy(val_hbm.at[pl.ds(base, _CPC)], val_v, insem).wait()

    def chunk(i, _):
        j = lax.rem(i, jnp.int32(_NBUF))
        row0 = (base + i) * jnp.int32(_CB)

        @pl.when(i >= jnp.int32(_NBUF))
        def _wait_prior():
            pltpu.make_async_copy(
                ring_ref.at[j],
                out_hbm.at[pl.ds(row0 - jnp.int32(_NBUF * _CB), _CB)],
                sems.at[j],
            ).wait()

        s = slot_v[i, 0, :]
        v = val_v[i, 0, :]
        iota = lax.broadcasted_iota(jnp.int32, (_CB, _SLOTS), 1)
        ring_ref[j] = jnp.where(iota == s[:, None], v[:, None], 0.0)

        pltpu.make_async_copy(
            ring_ref.at[j],
            out_hbm.at[pl.ds(row0, _CB)],
            sems.at[j],
        ).start()
        return jnp.int32(0)

    lax.fori_loop(jnp.int32(0), jnp.int32(_CPC), chunk, jnp.int32(0))

    def drain(i, _):
        j = lax.rem(i, jnp.int32(_NBUF))
        pltpu.make_async_copy(
            ring_ref.at[j],
            out_hbm.at[pl.ds((base + i) * jnp.int32(_CB), _CB)],
            sems.at[j],
        ).wait()
        return jnp.int32(0)

    lax.fori_loop(jnp.int32(_CPC - _NBUF), jnp.int32(_CPC), drain,
                  jnp.int32(0))


def kernel(input_tensor, values):
    x = input_tensor.astype(jnp.int32)
    x1 = (x >> 10).astype(jnp.int32)
    x0 = (x & 1023).astype(jnp.int32)
    t0 = jnp.asarray(_T0)
    t1 = jnp.asarray(_T1)
    dep = jnp.arange(_DEP)[:, None]
    s = t1[dep, x1[None, :]] + t0[dep, x0[None, :]]
    r = jnp.where(s >= jnp.uint32(_PRIME), s - jnp.uint32(_PRIME), s)
    slot_k = (r.astype(jnp.int32)) % _SLOTS
    slot_row = slot_k.reshape(-1).reshape(_BATCH, _DEP).T.reshape(-1)
    val_row = values.astype(jnp.float32).reshape(_BATCH, _DEP).T.reshape(-1)

    mesh = pltpu.create_tensorcore_mesh("core", num_cores=_NCORE)
    run = pl.kernel(
        _onehot_stream_body,
        out_type=jax.ShapeDtypeStruct((_ROWS, _SLOTS), jnp.float32),
        mesh=mesh,
        scratch_types=[
            pltpu.VMEM((_CPC, 1, _CB), jnp.int32),
            pltpu.VMEM((_CPC, 1, _CB), jnp.float32),
            pltpu.VMEM((_NBUF, _CB, _SLOTS), jnp.float32),
            pltpu.SemaphoreType.DMA((_NBUF,)),
            pltpu.SemaphoreType.DMA,
        ],
    )
    out = run(slot_row.reshape(_NCHUNK, 1, _CB), val_row.reshape(_NCHUNK, 1, _CB))
    return out.reshape(_DEP, _BATCH, _SLOTS)
